# baseline (speedup 1.0000x reference)
import functools

import jax
import jax.numpy as jnp
from jax import lax
from jax.experimental import pallas as pl
from jax.experimental.pallas import tpu as pltpu
from jax.experimental.pallas import tpu_sc as plsc

B = 16384
ED = 64
EPS = 1e-5

NC = 2
NS = 16
NW = NC * NS
BPW = B // NW
CHUNK = 128
NCHUNK = BPW // CHUNK


def _gather_body(uid_hbm, mid_hbm, ut_hbm, mt_hbm, ue_out, me_out,
                 uidx_v, midx_v, urows_v, mrows_v, usem, msem):
    wid = lax.axis_index("s") * NC + lax.axis_index("c")
    base = wid * BPW
    pltpu.sync_copy(uid_hbm.at[pl.ds(wid * NCHUNK, NCHUNK)], uidx_v)
    pltpu.sync_copy(mid_hbm.at[pl.ds(wid * NCHUNK, NCHUNK)], midx_v)
    cps = []
    for j in range(NCHUNK):
        cps.append(pltpu.async_copy(
            ut_hbm.at[uidx_v.at[j]], urows_v.at[pl.ds(j * CHUNK, CHUNK)],
            usem))
        cps.append(pltpu.async_copy(
            mt_hbm.at[midx_v.at[j]], mrows_v.at[pl.ds(j * CHUNK, CHUNK)],
            msem))
    for cp in cps:
        cp.wait()
    pltpu.sync_copy(urows_v, ue_out.at[pl.ds(base, BPW)])
    pltpu.sync_copy(mrows_v, me_out.at[pl.ds(base, BPW)])


@functools.cache
def _make_gather():
    return pl.kernel(
        _gather_body,
        out_type=[
            jax.ShapeDtypeStruct((B, ED), jnp.float32),
            jax.ShapeDtypeStruct((B, ED), jnp.float32),
        ],
        mesh=plsc.VectorSubcoreMesh(
            core_axis_name="c", subcore_axis_name="s",
            num_cores=NC, num_subcores=NS),
        scratch_types=[
            pltpu.VMEM((NCHUNK, CHUNK), jnp.int32),
            pltpu.VMEM((NCHUNK, CHUNK), jnp.int32),
            pltpu.VMEM((BPW, ED), jnp.float32),
            pltpu.VMEM((BPW, ED), jnp.float32),
            pltpu.SemaphoreType.DMA,
            pltpu.SemaphoreType.DMA,
        ],
        compiler_params=pltpu.CompilerParams(use_tc_tiling_on_sc=False),
    )


def _bn(x, g, be):
    mu = jnp.mean(x, axis=0, keepdims=True)
    d = x - mu
    var = jnp.mean(d * d, axis=0, keepdims=True)
    return d * lax.rsqrt(var + EPS) * g + be


def _mlp_body(ue, me, w1a, w1b, b1, g1, be1, w2, b2, g2, be2,
              w3, b3, g3, be3, w4t, b4, out):
    f32 = jnp.float32
    x = (jnp.dot(ue[...], w1a[...], preferred_element_type=f32)
         + jnp.dot(me[...], w1b[...], preferred_element_type=f32) + b1[...])
    x = _bn(jnp.maximum(x, 0.0), g1[...], be1[...])
    x = jnp.dot(x, w2[...], preferred_element_type=f32) + b2[...]
    x = _bn(jnp.maximum(x, 0.0), g2[...], be2[...])
    x = jnp.dot(x, w3[...], preferred_element_type=f32) + b3[...]
    x = _bn(jnp.maximum(x, 0.0), g3[...], be3[...])
    logit = jnp.sum(x * w4t[...], axis=1) + b4[0]
    out[...] = jax.nn.sigmoid(logit) * 4.5 + 0.5


_mlp = pl.pallas_call(
    _mlp_body,
    out_shape=jax.ShapeDtypeStruct((B,), jnp.float32),
)


def kernel(user_ids, movie_ids, user_table, movie_table,
           W1, b1, g1, be1, W2, b2, g2, be2, W3, b3, g3, be3, W4, b4):
    uid = user_ids.astype(jnp.int32).reshape(NW * NCHUNK, CHUNK)
    mid = movie_ids.astype(jnp.int32).reshape(NW * NCHUNK, CHUNK)
    ue, me = _make_gather()(uid, mid, user_table, movie_table)
    return _mlp(ue, me, W1[:ED], W1[ED:],
                b1.reshape(1, -1), g1.reshape(1, -1), be1.reshape(1, -1),
                W2, b2.reshape(1, -1), g2.reshape(1, -1), be2.reshape(1, -1),
                W3, b3.reshape(1, -1), g3.reshape(1, -1), be3.reshape(1, -1),
                W4.reshape(1, -1), b4)


# speedup vs baseline: 1.0012x; 1.0012x over previous
import functools

import jax
import jax.numpy as jnp
from jax import lax
from jax.experimental import pallas as pl
from jax.experimental.pallas import tpu as pltpu
from jax.experimental.pallas import tpu_sc as plsc

B = 16384
ED = 64
EPS = 1e-5

NC = 2
NS = 16
NW = NC * NS
BPW = B // NW
CHUNK = 128
NCH = BPW // CHUNK


def _gather_body(uid_hbm, mid_hbm, ut_hbm, mt_hbm, ue_out, me_out,
                 uidx_v, midx_v, urows_v, mrows_v, usem, msem):
    wid = lax.axis_index("s") * NC + lax.axis_index("c")
    base = wid * BPW
    pltpu.sync_copy(uid_hbm.at[wid], uidx_v)
    pltpu.sync_copy(mid_hbm.at[wid], midx_v)
    copies = []
    for j in range(NCH):
        copies.append(pltpu.async_copy(
            ut_hbm.at[uidx_v.at[j]],
            urows_v.at[pl.ds(j * CHUNK, CHUNK)], usem))
        copies.append(pltpu.async_copy(
            mt_hbm.at[midx_v.at[j]],
            mrows_v.at[pl.ds(j * CHUNK, CHUNK)], msem))
    for c in copies:
        c.wait()
    pltpu.sync_copy(urows_v, ue_out.at[pl.ds(base, BPW)])
    pltpu.sync_copy(mrows_v, me_out.at[pl.ds(base, BPW)])


@functools.cache
def _make_gather():
    return pl.kernel(
        _gather_body,
        out_type=[
            jax.ShapeDtypeStruct((B, ED), jnp.float32),
            jax.ShapeDtypeStruct((B, ED), jnp.float32),
        ],
        mesh=plsc.VectorSubcoreMesh(
            core_axis_name="c", subcore_axis_name="s",
            num_cores=NC, num_subcores=NS),
        compiler_params=pltpu.CompilerParams(use_tc_tiling_on_sc=False),
        scratch_types=[
            pltpu.VMEM((NCH, CHUNK), jnp.int32),
            pltpu.VMEM((NCH, CHUNK), jnp.int32),
            pltpu.VMEM((BPW, ED), jnp.float32),
            pltpu.VMEM((BPW, ED), jnp.float32),
            pltpu.SemaphoreType.DMA,
            pltpu.SemaphoreType.DMA,
        ],
    )


def _bn(x, g, be):
    mu = jnp.mean(x, axis=0, keepdims=True)
    d = x - mu
    var = jnp.mean(d * d, axis=0, keepdims=True)
    return d * lax.rsqrt(var + EPS) * g + be


def _mlp_body(ue, me, w1a, w1b, b1, g1, be1, w2, b2, g2, be2,
              w3, b3, g3, be3, w4, b4, out):
    f32 = jnp.float32
    x = (jnp.dot(ue[...], w1a[...], preferred_element_type=f32)
         + jnp.dot(me[...], w1b[...], preferred_element_type=f32)
         + b1[...])
    x = _bn(jnp.maximum(x, 0.0), g1[...], be1[...])
    x = jnp.dot(x, w2[...], preferred_element_type=f32) + b2[...]
    x = _bn(jnp.maximum(x, 0.0), g2[...], be2[...])
    x = jnp.dot(x, w3[...], preferred_element_type=f32) + b3[...]
    x = _bn(jnp.maximum(x, 0.0), g3[...], be3[...])
    logit = jnp.dot(x, w4[...], preferred_element_type=f32) + b4[...]
    out[...] = jax.nn.sigmoid(logit) * 4.5 + 0.5


_mlp = pl.pallas_call(
    _mlp_body,
    out_shape=jax.ShapeDtypeStruct((B, 1), jnp.float32),
)


def kernel(user_ids, movie_ids, user_table, movie_table,
           W1, b1, g1, be1, W2, b2, g2, be2, W3, b3, g3, be3, W4, b4):
    uid = user_ids.astype(jnp.int32).reshape(NW, NCH, CHUNK)
    mid = movie_ids.astype(jnp.int32).reshape(NW, NCH, CHUNK)
    ue, me = _make_gather()(uid, mid, user_table, movie_table)
    row = lambda v: v.reshape(1, -1)
    out = _mlp(ue, me, W1[:ED], W1[ED:],
               row(b1), row(g1), row(be1),
               W2, row(b2), row(g2), row(be2),
               W3, row(b3), row(g3), row(be3),
               W4, b4.reshape(1, 1))
    return out.reshape(B)
